# baseline (speedup 1.0000x reference)
import functools

import jax
import jax.numpy as jnp
import numpy as np
from jax import lax
from jax.experimental import pallas as pl
from jax.experimental.pallas import tpu as pltpu
from jax.experimental.pallas import tpu_sc as plsc

_GRID = 32
_D = 96
_N_S = _GRID * _GRID
_ND = _N_S // 2
_NSRC = _N_S // 2
_KEEP_RATE = 0.7
_ALPHA = 0.7
_R = int(_N_S * (1.0 - _KEEP_RATE))
_NUNM = _NSRC - _R
_NPAD = 256
_NOUT = 1 + _ND + _NUNM
_HI = lax.Precision.HIGHEST
_BS = 2
_NW = 32


def _build_low_basis():
    h = w = _GRID
    y, x = np.meshgrid(np.arange(h), np.arange(w), indexing="ij")
    cols = [np.full((h, w), 1.0 / np.sqrt(h * w))]
    for dy, dx in [(0, 1), (0, 2), (1, 0), (2, 0), (1, 1), (1, -1)]:
        ph = 2.0 * np.pi * (dy * y + dx * x) / h
        cols.append(np.sqrt(2.0 / (h * w)) * np.cos(ph))
        cols.append(np.sqrt(2.0 / (h * w)) * np.sin(ph))
    u = np.stack([c.reshape(-1) for c in cols], axis=1)
    u = np.concatenate([u, np.zeros((h * w, 3))], axis=1)
    return u.astype(np.float32)


_U = _build_low_basis()
_UD = np.ascontiguousarray(_U[0::2])
_US = np.ascontiguousarray(_U[1::2])


def _dot(a, b, ca, cb, precision=_HI):
    return lax.dot_general(a, b, (((ca,), (cb,)), ((), ())),
                           preferred_element_type=jnp.float32,
                           precision=precision)


def _one_sample(cls_row, xd, xs, ud, us, bglob):

    coef = _dot(ud, xd, 0, 0) + _dot(us, xs, 0, 0)
    hfd = xd - _dot(ud, coef, 1, 0)
    hfs = xs - _dot(us, coef, 1, 0)
    ed = jnp.sqrt(jnp.sum(hfd * hfd, axis=1, keepdims=True))
    es = jnp.sqrt(jnp.sum(hfs * hfs, axis=1, keepdims=True))
    pmin = jnp.minimum(jnp.min(ed), jnp.min(es))
    pmax = jnp.maximum(jnp.max(ed), jnp.max(es))
    inv = 1.0 / (pmax - pmin + 1e-6)
    phid_c = (ed - pmin) * inv
    phis_c = (es - pmin) * inv
    phid_r = phid_c.T

    nd = xd / jnp.maximum(jnp.sqrt(jnp.sum(xd * xd, axis=1, keepdims=True)),
                          1e-12)
    ns = xs / jnp.maximum(jnp.sqrt(jnp.sum(xs * xs, axis=1, keepdims=True)),
                          1e-12)
    nt = 4
    ts = _NSRC // nt
    jcol_t = lax.broadcasted_iota(jnp.int32, (ts, _ND), 1)
    nm_parts, idx_parts = [], []
    for t in range(nt):
        sl = slice(t * ts, (t + 1) * ts)
        sim_t = _dot(ns[sl], nd, 1, 1, precision=None)
        adj_t = sim_t * (1.0 - _ALPHA * jnp.maximum(phis_c[sl], phid_r))
        nm_t = jnp.max(adj_t, axis=1, keepdims=True)
        idx_t = jnp.min(jnp.where(adj_t == nm_t, jcol_t, _ND), axis=1,
                        keepdims=True)
        nm_parts.append(nm_t)
        idx_parts.append(idx_t)
    nm_c = jnp.concatenate(nm_parts, axis=0)
    idx_c = jnp.concatenate(idx_parts, axis=0)
    nm_r = nm_c.T

    irow_t = lax.broadcasted_iota(jnp.int32, (_NSRC, ts), 0)
    jcol_s = lax.broadcasted_iota(jnp.int32, (_NSRC, ts), 1)
    rank = jnp.zeros((_NSRC, 1), jnp.float32)
    for t in range(nt):
        sl = slice(t * ts, (t + 1) * ts)
        vj = nm_r[:, sl]
        better = (vj > nm_c) | ((vj == nm_c) & ((t * ts + jcol_s) < irow_t))
        rank = rank + jnp.sum(better.astype(jnp.float32), axis=1,
                              keepdims=True)
    mf_c = (rank < (_R - 0.5)).astype(jnp.float32)
    mf_r = mf_c.T
    keep_r = 1.0 - mf_r

    addv = jnp.zeros((_ND, _D), jnp.float32)
    cnt_r = jnp.zeros((1, _ND), jnp.float32)
    for t in range(nt):
        sl = slice(t * ts, (t + 1) * ts)
        sel_t = jnp.where(idx_c[sl] == jcol_t, mf_c[sl], 0.0)
        addv = addv + _dot(sel_t, xs[sl], 0, 0)
        cnt_r = cnt_r + jnp.sum(sel_t, axis=0, keepdims=True)
    dst_out = (xd + addv) / (1.0 + cnt_r.T)

    pos_c = jnp.zeros((_NSRC, 1), jnp.float32)
    for t in range(nt):
        sl = slice(t * ts, (t + 1) * ts)
        contrib = jnp.where((t * ts + jcol_s) < irow_t, keep_r[:, sl], 0.0)
        pos_c = pos_c + jnp.sum(contrib, axis=1, keepdims=True)
    posi_r = pos_c.astype(jnp.int32).T
    ival_c = lax.broadcasted_iota(jnp.int32, (_NSRC, 1), 0).astype(jnp.float32)
    g = jnp.zeros((_NPAD, 1), jnp.float32)
    prow_t = lax.broadcasted_iota(jnp.int32, (_NPAD, ts), 0)
    for t in range(nt):
        sl = slice(t * ts, (t + 1) * ts)
        gat_t = jnp.where((prow_t == posi_r[:, sl]) & (keep_r[:, sl] > 0.5),
                          1.0, 0.0)
        g = g + _dot(gat_t, ival_c[sl], 1, 0)
    gidx = g.astype(jnp.int32) + bglob * _NSRC

    return jnp.concatenate([cls_row, dst_out], axis=0), gidx


def _body(cls_ref, xd_ref, xs_ref, ud_ref, us_ref, out_ref, gl_ref):
    ud = ud_ref[...]
    us = us_ref[...]
    for s in range(_BS):
        bglob = pl.program_id(0) * _BS + s
        rows, gidx = _one_sample(cls_ref[s], xd_ref[s], xs_ref[s], ud, us,
                                 bglob)
        out_ref[s] = rows
        gl_ref[s] = gidx


def _tc_stage(cls_tok, xd, xs):
    b = xd.shape[0]
    return pl.pallas_call(
        _body,
        grid=(b // _BS,),
        in_specs=[
            pl.BlockSpec((_BS, 1, _D), lambda i: (i, 0, 0)),
            pl.BlockSpec((_BS, _ND, _D), lambda i: (i, 0, 0)),
            pl.BlockSpec((_BS, _NSRC, _D), lambda i: (i, 0, 0)),
            pl.BlockSpec((_ND, 16), lambda i: (0, 0)),
            pl.BlockSpec((_NSRC, 16), lambda i: (0, 0)),
        ],
        out_specs=[
            pl.BlockSpec((_BS, 1 + _ND, _D), lambda i: (i, 0, 0)),
            pl.BlockSpec((_BS, _NPAD, 1), lambda i: (i, 0, 0)),
        ],
        out_shape=[
            jax.ShapeDtypeStruct((b, 1 + _ND, _D), jnp.float32),
            jax.ShapeDtypeStruct((b, _NPAD, 1), jnp.int32),
        ],
    )(cls_tok, xd, xs, _UD, _US)


def _sc_compact(xs_flat, glist):
    b = glist.shape[0]
    spw = b // _NW
    mesh = plsc.VectorSubcoreMesh(core_axis_name="c", subcore_axis_name="s")

    @functools.partial(
        pl.kernel, mesh=mesh,
        compiler_params=pltpu.CompilerParams(use_tc_tiling_on_sc=False),
        out_type=jax.ShapeDtypeStruct((b, _NPAD, _D), jnp.float32),
        scratch_types=[
            pltpu.VMEM((2, 128), jnp.int32),
            pltpu.VMEM((_NPAD, _D), jnp.float32),
            pltpu.SemaphoreType.DMA,
        ],
    )
    def k(xs_hbm, gl_hbm, out_hbm, idx_v, rows_v, sem):
        wid = lax.axis_index("s") * 2 + lax.axis_index("c")
        for t in range(spw):
            sb = wid * spw + t
            pltpu.sync_copy(gl_hbm.at[sb], idx_v)
            pltpu.async_copy(xs_hbm.at[idx_v.at[0]],
                             rows_v.at[pl.ds(0, 128)], sem).wait()
            pltpu.async_copy(xs_hbm.at[idx_v.at[1]],
                             rows_v.at[pl.ds(128, 128)], sem).wait()
            pltpu.sync_copy(rows_v, out_hbm.at[sb])

    return k(xs_flat, glist)


@functools.partial(jax.jit, static_argnums=())
def kernel(tokens):
    b = tokens.shape[0]
    cls_tok = tokens[:, :1]
    t3 = tokens[:, 1:].reshape(b, _ND, 2, _D)
    xd = t3[:, :, 0]
    xs = t3[:, :, 1]
    out1, glist = _tc_stage(cls_tok, xd, xs)
    unm = _sc_compact(xs.reshape(b * _NSRC, _D),
                      glist.reshape(b, 2, 128))
    return jnp.concatenate([out1, unm[:, :_NUNM]], axis=1)
